# baseline (speedup 1.0000x reference)
import functools

import jax
import jax.numpy as jnp
from jax import lax
from jax.experimental import pallas as pl
from jax.experimental.pallas import tpu as pltpu
from jax.experimental.pallas import tpu_sc as plsc

D_MODEL = 768
SEQ = 512
N_CORES = 2
N_SUBCORES = 16
N_WORKERS = N_CORES * N_SUBCORES
I_PER_W = SEQ // N_WORKERS


BLK = SEQ * D_MODEL


def _sc_copy(emb_rev_hbm, out_hbm, sem):
    wid = lax.axis_index("s") * N_CORES + lax.axis_index("c")
    base_i = wid * I_PER_W
    copies = []
    for t in range(I_PER_W):
        i = base_i + t
        copies.append(pltpu.async_copy(
            emb_rev_hbm.at[pl.ds((SEQ - i) * D_MODEL, BLK)],
            out_hbm.at[pl.ds(i * BLK, BLK)],
            sem,
        ))
    for c in copies:
        c.wait()


def kernel(seq_len, emb):
    del seq_len
    emb_rev = emb[::-1].reshape(-1)
    mesh = plsc.VectorSubcoreMesh(core_axis_name="c", subcore_axis_name="s")
    out_flat = pl.kernel(
        _sc_copy,
        mesh=mesh,
        out_type=jax.ShapeDtypeStruct((SEQ * SEQ * D_MODEL,), jnp.float32),
        scratch_types=[pltpu.SemaphoreType.DMA],
    )(emb_rev)
    return out_flat.reshape(SEQ, SEQ, D_MODEL)


# speedup vs baseline: 20.2284x; 20.2284x over previous
import functools

import jax
import jax.numpy as jnp
from jax import lax
from jax.experimental import pallas as pl
from jax.experimental.pallas import tpu as pltpu
from jax.experimental.pallas import tpu_sc as plsc

D_MODEL = 768
SEQ = 512
N_CORES = 2
N_SUBCORES = 16
N_WORKERS = N_CORES * N_SUBCORES
I_PER_W = SEQ // N_WORKERS


BLK = SEQ * D_MODEL


def _sc_copy(emb_rev_hbm, out_hbm, table_spmem, sem):
    wid = lax.axis_index("s") * N_CORES + lax.axis_index("c")
    @pl.when(lax.axis_index("s") == 0)
    def _stage():
        pltpu.sync_copy(emb_rev_hbm, table_spmem)

    plsc.subcore_barrier()

    base_i = wid * I_PER_W
    copies = []
    for t in range(I_PER_W):
        i = base_i + t
        copies.append(pltpu.async_copy(
            table_spmem.at[pl.ds((SEQ - i) * D_MODEL, BLK)],
            out_hbm.at[pl.ds(i * BLK, BLK)],
            sem,
        ))
    for c in copies:
        c.wait()


def kernel(seq_len, emb):
    del seq_len
    emb_rev = emb[::-1].reshape(-1)
    mesh = plsc.VectorSubcoreMesh(core_axis_name="c", subcore_axis_name="s")
    out_flat = pl.kernel(
        _sc_copy,
        mesh=mesh,
        out_type=jax.ShapeDtypeStruct((SEQ * SEQ * D_MODEL,), jnp.float32),
        scratch_types=[
            pltpu.VMEM_SHARED((1025 * D_MODEL,), jnp.float32),
            pltpu.SemaphoreType.DMA,
        ],
    )(emb_rev)
    return out_flat.reshape(SEQ, SEQ, D_MODEL)


# speedup vs baseline: 22.3088x; 1.1028x over previous
import functools

import jax
import jax.numpy as jnp
from jax import lax
from jax.experimental import pallas as pl
from jax.experimental.pallas import tpu as pltpu
from jax.experimental.pallas import tpu_sc as plsc

D_MODEL = 768
SEQ = 512
N_CORES = 2
N_SUBCORES = 16
N_WORKERS = N_CORES * N_SUBCORES
I_PER_W = SEQ // N_WORKERS
BLK = SEQ * D_MODEL

WIN = SEQ + I_PER_W
CH = 48
N_CHUNKS = WIN // CH


def _sc_copy(emb_rev_hbm, out_hbm, buf0, buf1, gsem, ssem0, ssem1):
    wid = lax.axis_index("s") * N_CORES + lax.axis_index("c")
    base_i = wid * I_PER_W
    win0 = (SEQ - I_PER_W) - base_i + I_PER_W * 0
    bufs = (buf0, buf1)
    ssems = (ssem0, ssem1)

    def gather(p):
        return pltpu.async_copy(
            emb_rev_hbm.at[pl.ds((win0 + p * CH) * D_MODEL, CH * D_MODEL)],
            bufs[p % 2],
            gsem,
        )

    pending_scatters = {}
    g = gather(0)
    for p in range(N_CHUNKS):
        b = p % 2
        g.wait()
        handles = []
        for t in range(I_PER_W):
            s0 = max(p * CH, I_PER_W - t)
            s1 = min((p + 1) * CH, WIN - t)
            if s1 <= s0:
                continue
            dst_row = s0 - (I_PER_W - t)
            handles.append(pltpu.async_copy(
                bufs[b].at[pl.ds((s0 - p * CH) * D_MODEL, (s1 - s0) * D_MODEL)],
                out_hbm.at[pl.ds((base_i + t) * BLK + dst_row * D_MODEL,
                                 (s1 - s0) * D_MODEL)],
                ssems[b],
            ))
        pending_scatters[p] = handles
        if p + 1 < N_CHUNKS:
            if p - 1 >= 0:
                for h in pending_scatters.pop(p - 1):
                    h.wait()
            g = gather(p + 1)
    for hs in pending_scatters.values():
        for h in hs:
            h.wait()


def kernel(seq_len, emb):
    del seq_len
    emb_rev = emb[::-1].reshape(-1)
    mesh = plsc.VectorSubcoreMesh(core_axis_name="c", subcore_axis_name="s")
    out_flat = pl.kernel(
        _sc_copy,
        mesh=mesh,
        out_type=jax.ShapeDtypeStruct((SEQ * SEQ * D_MODEL,), jnp.float32),
        scratch_types=[
            pltpu.VMEM((CH * D_MODEL,), jnp.float32),
            pltpu.VMEM((CH * D_MODEL,), jnp.float32),
            pltpu.SemaphoreType.DMA,
            pltpu.SemaphoreType.DMA,
            pltpu.SemaphoreType.DMA,
        ],
    )(emb_rev)
    return out_flat.reshape(SEQ, SEQ, D_MODEL)
